# baseline (speedup 1.0000x reference)
import functools

import jax
import jax.numpy as jnp
from jax.experimental import pallas as pl

_HIGH = jax.lax.Precision.HIGHEST
_INTERPRET = False


def _router_body(logits_ref, dest_ref, scale_ref, aux_ref, *, T, E, K, CAP, TRASH):
    logits = logits_ref[...]
    iota_e = jax.lax.broadcasted_iota(jnp.int32, (T, E), 1)
    m1 = jnp.max(logits, axis=1, keepdims=True)
    a1 = jnp.min(jnp.where(logits == m1, iota_e, E), axis=1, keepdims=True)
    neg = jnp.float32(-jnp.inf)
    masked = jnp.where(iota_e == a1, neg, logits)
    m2 = jnp.max(masked, axis=1, keepdims=True)
    a2 = jnp.min(jnp.where(masked == m2, iota_e, E), axis=1, keepdims=True)
    z = jnp.exp(m2 - m1)
    g1 = 1.0 / (1.0 + z)
    g2 = z / (1.0 + z)
    p = jnp.exp(logits - m1)
    probs = p / jnp.sum(p, axis=1, keepdims=True)
    imp = jnp.mean(probs, axis=0, keepdims=True)
    oh1 = (iota_e == a1).astype(jnp.float32)
    oh2 = (iota_e == a2).astype(jnp.float32)
    c = oh1 + oh2
    counts = jnp.sum(c, axis=0, keepdims=True)
    aux_ref[...] = jnp.reshape(E * jnp.sum(imp * counts) / (T * K), (1, 1))
    s = c
    sh = 1
    while sh < T:
        s = s + jnp.concatenate(
            [jnp.zeros((sh, E), jnp.float32), s[: T - sh]], axis=0)
        sh *= 2
    cex = s - c
    pos1 = jnp.sum(cex * oh1, axis=1, keepdims=True).astype(jnp.int32)
    pos2 = jnp.sum(cex * oh2, axis=1, keepdims=True).astype(jnp.int32)
    keep1 = pos1 < CAP
    keep2 = pos2 < CAP
    d1 = jnp.where(keep1, a1 * CAP + jnp.minimum(pos1, CAP - 1), TRASH)
    d2 = jnp.where(keep2, a2 * CAP + jnp.minimum(pos2, CAP - 1), TRASH)
    dest_ref[...] = jnp.concatenate([d1, d2], axis=1)
    scale_ref[...] = jnp.concatenate(
        [jnp.where(keep1, g1, 0.0), jnp.where(keep2, g2, 0.0)], axis=1)


def _ffn_body(buf_ref, w1_ref, w2_ref, y_ref):
    f = pl.program_id(1)
    h = jax.nn.gelu(jax.lax.dot_general(
        buf_ref[...], w1_ref[0], (((1,), (0,)), ((), ())),
        preferred_element_type=jnp.float32, precision=_HIGH))
    contrib = jax.lax.dot_general(
        h, w2_ref[0], (((1,), (0,)), ((), ())),
        preferred_element_type=jnp.float32, precision=_HIGH)

    @pl.when(f == 0)
    def _():
        y_ref[...] = contrib

    @pl.when(f != 0)
    def _():
        y_ref[...] = y_ref[...] + contrib


def kernel(hidden_states, w_router, w1, w2):
    T, D = hidden_states.shape
    E = w_router.shape[1]
    F = w1.shape[2]
    K = 2
    CAP = int(T * K / E * 1.25)
    TRASH = E * CAP
    NPAD = 8

    logits = hidden_states @ w_router

    dest, scale, aux = pl.pallas_call(
        functools.partial(_router_body, T=T, E=E, K=K, CAP=CAP, TRASH=TRASH),
        out_shape=(
            jax.ShapeDtypeStruct((T, K), jnp.int32),
            jax.ShapeDtypeStruct((T, K), jnp.float32),
            jax.ShapeDtypeStruct((1, 1), jnp.float32),
        ),
        interpret=_INTERPRET,
    )(logits)

    dest_f = dest.reshape(T * K)
    scale_f = scale.reshape(T * K)

    hs_dup = jnp.repeat(hidden_states, K, axis=0)
    buf = jnp.zeros((TRASH + NPAD, D), jnp.float32).at[dest_f].set(hs_dup)

    FB = min(F, 1024)
    NF = F // FB
    y = pl.pallas_call(
        _ffn_body,
        grid=(E, NF),
        in_specs=[
            pl.BlockSpec((CAP, D), lambda e, f: (e, 0)),
            pl.BlockSpec((1, D, FB), lambda e, f: (e, 0, f)),
            pl.BlockSpec((1, FB, D), lambda e, f: (e, f, 0)),
        ],
        out_specs=pl.BlockSpec((CAP, D), lambda e, f: (e, 0)),
        out_shape=jax.ShapeDtypeStruct((E * CAP, D), jnp.float32),
        interpret=_INTERPRET,
    )(buf, w1, w2)

    ypad = jnp.concatenate([y, jnp.zeros((NPAD, D), jnp.float32)], axis=0)
    gathered = ypad[dest_f]
    contrib = jnp.where((scale_f != 0.0)[:, None], gathered * scale_f[:, None], 0.0)
    out = jnp.sum(contrib.reshape(T, K, D), axis=1)
    return out, aux.reshape(())


# speedup vs baseline: 2.5787x; 2.5787x over previous
import functools

import jax
import jax.numpy as jnp
from jax.experimental import pallas as pl

_HIGH = jax.lax.Precision.HIGHEST
_INTERPRET = False


def _router_body(logits_ref, dest_ref, scale_ref, aux_ref, *, T, E, K, CAP, TRASH):
    logits = logits_ref[...]
    iota_e = jax.lax.broadcasted_iota(jnp.int32, (T, E), 1)
    m1 = jnp.max(logits, axis=1, keepdims=True)
    a1 = jnp.min(jnp.where(logits == m1, iota_e, E), axis=1, keepdims=True)
    neg = jnp.float32(-jnp.inf)
    masked = jnp.where(iota_e == a1, neg, logits)
    m2 = jnp.max(masked, axis=1, keepdims=True)
    a2 = jnp.min(jnp.where(masked == m2, iota_e, E), axis=1, keepdims=True)
    z = jnp.exp(m2 - m1)
    g1 = 1.0 / (1.0 + z)
    g2 = z / (1.0 + z)
    p = jnp.exp(logits - m1)
    probs = p / jnp.sum(p, axis=1, keepdims=True)
    imp = jnp.mean(probs, axis=0, keepdims=True)
    oh1 = (iota_e == a1).astype(jnp.float32)
    oh2 = (iota_e == a2).astype(jnp.float32)
    c = oh1 + oh2
    counts = jnp.sum(c, axis=0, keepdims=True)
    aux_ref[...] = jnp.reshape(E * jnp.sum(imp * counts) / (T * K), (1, 1))
    s = c
    sh = 1
    while sh < T:
        s = s + jnp.concatenate(
            [jnp.zeros((sh, E), jnp.float32), s[: T - sh]], axis=0)
        sh *= 2
    cex = s - c
    pos1 = jnp.sum(cex * oh1, axis=1, keepdims=True).astype(jnp.int32)
    pos2 = jnp.sum(cex * oh2, axis=1, keepdims=True).astype(jnp.int32)
    keep1 = pos1 < CAP
    keep2 = pos2 < CAP
    d1 = jnp.where(keep1, a1 * CAP + jnp.minimum(pos1, CAP - 1), TRASH)
    d2 = jnp.where(keep2, a2 * CAP + jnp.minimum(pos2, CAP - 1), TRASH)
    dest_ref[...] = jnp.concatenate([d1, d2], axis=1)
    scale_ref[...] = jnp.concatenate(
        [jnp.where(keep1, g1, 0.0), jnp.where(keep2, g2, 0.0)], axis=1)


def _ffn_body(buf_ref, w1_ref, w2_ref, y_ref):
    f = pl.program_id(1)
    h = jax.nn.gelu(jax.lax.dot_general(
        buf_ref[...], w1_ref[0], (((1,), (0,)), ((), ())),
        preferred_element_type=jnp.float32))
    contrib = jax.lax.dot_general(
        h, w2_ref[0], (((1,), (0,)), ((), ())),
        preferred_element_type=jnp.float32)

    @pl.when(f == 0)
    def _():
        y_ref[...] = contrib

    @pl.when(f != 0)
    def _():
        y_ref[...] = y_ref[...] + contrib


def kernel(hidden_states, w_router, w1, w2):
    T, D = hidden_states.shape
    E = w_router.shape[1]
    F = w1.shape[2]
    K = 2
    CAP = int(T * K / E * 1.25)
    TRASH = E * CAP
    NPAD = 8

    logits = hidden_states @ w_router

    dest, scale, aux = pl.pallas_call(
        functools.partial(_router_body, T=T, E=E, K=K, CAP=CAP, TRASH=TRASH),
        out_shape=(
            jax.ShapeDtypeStruct((T, K), jnp.int32),
            jax.ShapeDtypeStruct((T, K), jnp.float32),
            jax.ShapeDtypeStruct((1, 1), jnp.float32),
        ),
        interpret=_INTERPRET,
    )(logits)

    dest_f = dest.reshape(T * K)
    scale_f = scale.reshape(T * K)

    hs_dup = jnp.repeat(hidden_states, K, axis=0)
    buf = jnp.zeros((TRASH + NPAD, D), jnp.float32).at[dest_f].set(hs_dup)

    FB = min(F, 1024)
    NF = F // FB
    y = pl.pallas_call(
        _ffn_body,
        grid=(E, NF),
        in_specs=[
            pl.BlockSpec((CAP, D), lambda e, f: (e, 0)),
            pl.BlockSpec((1, D, FB), lambda e, f: (e, 0, f)),
            pl.BlockSpec((1, FB, D), lambda e, f: (e, f, 0)),
        ],
        out_specs=pl.BlockSpec((CAP, D), lambda e, f: (e, 0)),
        out_shape=jax.ShapeDtypeStruct((E * CAP, D), jnp.float32),
        interpret=_INTERPRET,
    )(buf, w1, w2)

    ypad = jnp.concatenate([y, jnp.zeros((NPAD, D), jnp.float32)], axis=0)
    gathered = ypad[dest_f]
    contrib = jnp.where((scale_f != 0.0)[:, None], gathered * scale_f[:, None], 0.0)
    out = jnp.sum(contrib.reshape(T, K, D), axis=1)
    return out, aux.reshape(())


# speedup vs baseline: 3.8775x; 1.5037x over previous
import functools

import jax
import jax.numpy as jnp
from jax import lax
from jax.experimental import pallas as pl
from jax.experimental.pallas import tpu as pltpu
from jax.experimental.pallas import tpu_sc as plsc

_NC = 2
_NS = 16
_NW = _NC * _NS


def _router_body(logits_ref, dest_ref, scale_ref, aux_ref, *, T, E, K, CAP, TRASH):
    logits = logits_ref[...]
    iota_e = jax.lax.broadcasted_iota(jnp.int32, (T, E), 1)
    m1 = jnp.max(logits, axis=1, keepdims=True)
    a1 = jnp.min(jnp.where(logits == m1, iota_e, E), axis=1, keepdims=True)
    neg = jnp.float32(-jnp.inf)
    masked = jnp.where(iota_e == a1, neg, logits)
    m2 = jnp.max(masked, axis=1, keepdims=True)
    a2 = jnp.min(jnp.where(masked == m2, iota_e, E), axis=1, keepdims=True)
    z = jnp.exp(m2 - m1)
    g1 = 1.0 / (1.0 + z)
    g2 = z / (1.0 + z)
    p = jnp.exp(logits - m1)
    probs = p / jnp.sum(p, axis=1, keepdims=True)
    imp = jnp.mean(probs, axis=0, keepdims=True)
    oh1 = (iota_e == a1).astype(jnp.float32)
    oh2 = (iota_e == a2).astype(jnp.float32)
    c = oh1 + oh2
    counts = jnp.sum(c, axis=0, keepdims=True)
    aux_ref[...] = jnp.reshape(E * jnp.sum(imp * counts) / (T * K), (1, 1))
    s = c
    sh = 1
    while sh < T:
        s = s + jnp.concatenate(
            [jnp.zeros((sh, E), jnp.float32), s[: T - sh]], axis=0)
        sh *= 2
    cex = s - c
    pos1 = jnp.sum(cex * oh1, axis=1, keepdims=True).astype(jnp.int32)
    pos2 = jnp.sum(cex * oh2, axis=1, keepdims=True).astype(jnp.int32)
    keep1 = pos1 < CAP
    keep2 = pos2 < CAP
    d1 = jnp.where(keep1, a1 * CAP + jnp.minimum(pos1, CAP - 1), TRASH)
    d2 = jnp.where(keep2, a2 * CAP + jnp.minimum(pos2, CAP - 1), TRASH)
    dest_ref[...] = jnp.concatenate([d1, d2], axis=1)
    scale_ref[...] = jnp.concatenate(
        [jnp.where(keep1, g1, 0.0), jnp.where(keep2, g2, 0.0)], axis=1)


def _dispatch_body(hs_hbm, d0_hbm, d1_hbm, buf_hbm, rows_v, idx0_v, idx1_v, sem,
                   *, tpw, ch):
    wid = lax.axis_index("s") * _NC + lax.axis_index("c")
    base = wid * tpw
    pltpu.sync_copy(d0_hbm.at[wid], idx0_v)
    pltpu.sync_copy(d1_hbm.at[wid], idx1_v)
    for j in range(tpw // ch):
        pltpu.sync_copy(hs_hbm.at[pl.ds(base + j * ch, ch)], rows_v)
        pltpu.async_copy(rows_v, buf_hbm.at[idx0_v.at[j]], sem).wait()
        pltpu.async_copy(rows_v, buf_hbm.at[idx1_v.at[j]], sem).wait()


def _combine_body(y_hbm, d0_hbm, d1_hbm, g0_hbm, g1_hbm, rows_v, idx0_v, idx1_v,
                  sem, *, tpw, ch):
    wid = lax.axis_index("s") * _NC + lax.axis_index("c")
    base = wid * tpw
    pltpu.sync_copy(d0_hbm.at[wid], idx0_v)
    pltpu.sync_copy(d1_hbm.at[wid], idx1_v)
    for j in range(tpw // ch):
        pltpu.async_copy(y_hbm.at[idx0_v.at[j]], rows_v, sem).wait()
        pltpu.sync_copy(rows_v, g0_hbm.at[pl.ds(base + j * ch, ch)])
        pltpu.async_copy(y_hbm.at[idx1_v.at[j]], rows_v, sem).wait()
        pltpu.sync_copy(rows_v, g1_hbm.at[pl.ds(base + j * ch, ch)])


def _ffn_body(buf_ref, w1_ref, w2_ref, y_ref):
    f = pl.program_id(1)
    h = jax.nn.gelu(jax.lax.dot_general(
        buf_ref[...], w1_ref[0], (((1,), (0,)), ((), ())),
        preferred_element_type=jnp.float32))
    contrib = jax.lax.dot_general(
        h, w2_ref[0], (((1,), (0,)), ((), ())),
        preferred_element_type=jnp.float32)

    @pl.when(f == 0)
    def _():
        y_ref[...] = contrib

    @pl.when(f != 0)
    def _():
        y_ref[...] = y_ref[...] + contrib


def _merge_body(g0_ref, g1_ref, s0_ref, s1_ref, out_ref):
    s0 = s0_ref[...]
    s1 = s1_ref[...]
    out_ref[...] = (
        jnp.where(s0 != 0.0, g0_ref[...] * s0, 0.0)
        + jnp.where(s1 != 0.0, g1_ref[...] * s1, 0.0))


def kernel(hidden_states, w_router, w1, w2):
    T, D = hidden_states.shape
    E = w_router.shape[1]
    F = w1.shape[2]
    K = 2
    CAP = int(T * K / E * 1.25)
    TRASH = E * CAP
    NPAD = 8
    NBUF = TRASH + NPAD

    logits = hidden_states @ w_router

    dest, scale, aux = pl.pallas_call(
        functools.partial(_router_body, T=T, E=E, K=K, CAP=CAP, TRASH=TRASH),
        out_shape=(
            jax.ShapeDtypeStruct((T, K), jnp.int32),
            jax.ShapeDtypeStruct((T, K), jnp.float32),
            jax.ShapeDtypeStruct((1, 1), jnp.float32),
        ),
    )(logits)

    TPW = T // _NW
    CH = min(TPW, 32)
    d0 = dest[:, 0].reshape(_NW, TPW // CH, CH)
    d1 = dest[:, 1].reshape(_NW, TPW // CH, CH)

    mesh = plsc.VectorSubcoreMesh(core_axis_name="c", subcore_axis_name="s")

    dispatch = functools.partial(
        pl.kernel,
        mesh=mesh,
        out_type=jax.ShapeDtypeStruct((NBUF, D), jnp.float32),
        scratch_types=[
            pltpu.VMEM((CH, D), jnp.float32),
            pltpu.VMEM((TPW // CH, CH), jnp.int32),
            pltpu.VMEM((TPW // CH, CH), jnp.int32),
            pltpu.SemaphoreType.DMA,
        ],
    )(functools.partial(_dispatch_body, tpw=TPW, ch=CH))
    buf = dispatch(hidden_states, d0, d1)

    FB = min(F, 1024)
    NF = F // FB
    y = pl.pallas_call(
        _ffn_body,
        grid=(E, NF),
        in_specs=[
            pl.BlockSpec((CAP, D), lambda e, f: (e, 0)),
            pl.BlockSpec((1, D, FB), lambda e, f: (e, 0, f)),
            pl.BlockSpec((1, FB, D), lambda e, f: (e, f, 0)),
        ],
        out_specs=pl.BlockSpec((CAP, D), lambda e, f: (e, 0)),
        out_shape=jax.ShapeDtypeStruct((NBUF, D), jnp.float32),
    )(buf, w1, w2)

    combine = functools.partial(
        pl.kernel,
        mesh=mesh,
        out_type=(
            jax.ShapeDtypeStruct((T, D), jnp.float32),
            jax.ShapeDtypeStruct((T, D), jnp.float32),
        ),
        scratch_types=[
            pltpu.VMEM((CH, D), jnp.float32),
            pltpu.VMEM((TPW // CH, CH), jnp.int32),
            pltpu.VMEM((TPW // CH, CH), jnp.int32),
            pltpu.SemaphoreType.DMA,
        ],
    )(functools.partial(_combine_body, tpw=TPW, ch=CH))
    g0, g1 = combine(y, d0, d1)

    RB = 256
    out = pl.pallas_call(
        _merge_body,
        grid=(T // RB,),
        in_specs=[
            pl.BlockSpec((RB, D), lambda i: (i, 0)),
            pl.BlockSpec((RB, D), lambda i: (i, 0)),
            pl.BlockSpec((RB, 1), lambda i: (i, 0)),
            pl.BlockSpec((RB, 1), lambda i: (i, 0)),
        ],
        out_specs=pl.BlockSpec((RB, D), lambda i: (i, 0)),
        out_shape=jax.ShapeDtypeStruct((T, D), jnp.float32),
    )(g0, g1, scale[:, 0:1], scale[:, 1:2])

    return out, aux.reshape(())
